# Initial kernel scaffold; baseline (speedup 1.0000x reference)
import functools

import jax
import jax.numpy as jnp
from jax import lax
from jax.experimental import pallas as pl
from jax.experimental.pallas import tpu as pltpu

try:
    from jax.experimental.pallas import tpu_sc as plsc
    _HAVE_SC = True
except ImportError:
    plsc = None
    _HAVE_SC = False

KNN = 16
_BIG = jnp.float32(3.0e38)


def _topk_body(xt_blk_ref, xt_full_ref, idx_ref, *, n_total, k):
    xr = xt_blk_ref[0]
    xf = xt_full_ref[0]
    rowsq = jnp.sum(xr * xr, axis=1, keepdims=True)
    colsq = jnp.sum(xf * xf, axis=1)[None, :]
    prod = lax.dot_general(
        xr, xf, (((1,), (1,)), ((), ())),
        preferred_element_type=jnp.float32)
    d = rowsq + colsq - 2.0 * prod
    d = jnp.maximum(d, jnp.float32(1e-12))

    iota = lax.broadcasted_iota(jnp.int32, d.shape, 1)
    nbig = jnp.int32(n_total)
    cols = []
    for t in range(k + 1):
        m = jnp.min(d, axis=1, keepdims=True)
        cand = jnp.where(d == m, iota, nbig)
        j = jnp.min(cand, axis=1, keepdims=True)
        if t > 0:
            cols.append(j)
        d = jnp.where(iota == j, _BIG, d)
    idx_ref[0] = jnp.concatenate(cols, axis=1)


def _topk_indices(xt, *, row_block, interpret=False):
    B, N, C = xt.shape
    grid = (B, N // row_block)
    return pl.pallas_call(
        functools.partial(_topk_body, n_total=N, k=KNN),
        grid=grid,
        in_specs=[
            pl.BlockSpec((1, row_block, C), lambda b, i: (b, i, 0)),
            pl.BlockSpec((1, N, C), lambda b, i: (b, 0, 0)),
        ],
        out_specs=pl.BlockSpec((1, row_block, KNN), lambda b, i: (b, i, 0)),
        out_shape=jax.ShapeDtypeStruct((B, N, KNN), jnp.int32),
        interpret=interpret,
    )(xt, xt)


def _sc_gather(x, idx_flat):
    B, C, N = x.shape
    NK = N * KNN
    info = plsc.get_sparse_core_info()
    NC, NS = info.num_cores, info.num_subcores
    NW = NC * NS
    wpb = NW // B
    cpw = C // wpb
    mesh = plsc.VectorSubcoreMesh(core_axis_name="c", subcore_axis_name="s")

    @functools.partial(
        pl.kernel,
        out_type=jax.ShapeDtypeStruct((B, 2 * C, NK), jnp.float32),
        mesh=mesh,
        scratch_types=[
            pltpu.VMEM((NK,), jnp.int32),
            pltpu.VMEM((N,), jnp.float32),
            pltpu.VMEM((NK,), jnp.float32),
            pltpu.VMEM((NK,), jnp.float32),
        ],
    )
    def sc_kernel(x_hbm, idx_hbm, out_hbm, idx_v, row_v, cbuf, nbuf):
        wid = lax.axis_index("s") * NC + lax.axis_index("c")
        b = wid // wpb
        c0 = (wid % wpb) * cpw
        pltpu.sync_copy(idx_hbm.at[b], idx_v)

        def chan(ch, _):
            c = c0 + ch
            pltpu.sync_copy(x_hbm.at[b, c], row_v)

            def point(i, _):
                iv = idx_v[pl.ds(i * KNN, KNN)]
                nb = plsc.load_gather(row_v, [iv])
                center = jnp.full((KNN,), row_v[i], jnp.float32)
                cbuf[pl.ds(i * KNN, KNN)] = center
                nbuf[pl.ds(i * KNN, KNN)] = nb - center
                return 0

            lax.fori_loop(0, N, point, 0)
            pltpu.sync_copy(cbuf, out_hbm.at[b, c])
            pltpu.sync_copy(nbuf, out_hbm.at[b, C + c])
            return 0

        lax.fori_loop(0, cpw, chan, 0)

    return sc_kernel(x, idx_flat)


def kernel(cloud):
    B, C, N = cloud.shape
    xt = jnp.transpose(cloud, (0, 2, 1))
    idx = _topk_indices(xt, row_block=256)
    idx_flat = idx.reshape(B, N * KNN)
    out = _sc_gather(cloud, idx_flat)
    return out.reshape(B, 2 * C, N, KNN)


# speedup vs baseline: 5.8023x; 5.8023x over previous
import functools

import jax
import jax.numpy as jnp
from jax import lax
from jax.experimental import pallas as pl
from jax.experimental.pallas import tpu as pltpu

try:
    from jax.experimental.pallas import tpu_sc as plsc
    _HAVE_SC = True
except ImportError:
    plsc = None
    _HAVE_SC = False

KNN = 16
_BIG = 3.0e38


def _topk_body(xt_blk_ref, xt_full_ref, idx_ref, *, n_total, k):
    xr = xt_blk_ref[0]
    xf = xt_full_ref[0]
    rowsq = jnp.sum(xr * xr, axis=1, keepdims=True)
    colsq = jnp.sum(xf * xf, axis=1)[None, :]
    prod = lax.dot_general(
        xr, xf, (((1,), (1,)), ((), ())),
        preferred_element_type=jnp.float32)
    d = rowsq + colsq - 2.0 * prod
    d = jnp.maximum(d, jnp.float32(1e-12))

    iota = lax.broadcasted_iota(jnp.int32, d.shape, 1)
    nbig = jnp.int32(n_total)
    cols = []
    for t in range(k + 1):
        m = jnp.min(d, axis=1, keepdims=True)
        cand = jnp.where(d == m, iota, nbig)
        j = jnp.min(cand, axis=1, keepdims=True)
        if t > 0:
            cols.append(j)
        d = jnp.where(iota == j, jnp.float32(_BIG), d)
    idx_ref[0] = jnp.concatenate(cols, axis=1)


def _topk_indices(xt, *, row_block, interpret=False):
    B, N, C = xt.shape
    grid = (B, N // row_block)
    return pl.pallas_call(
        functools.partial(_topk_body, n_total=N, k=KNN),
        grid=grid,
        in_specs=[
            pl.BlockSpec((1, row_block, C), lambda b, i: (b, i, 0)),
            pl.BlockSpec((1, N, C), lambda b, i: (b, 0, 0)),
        ],
        out_specs=pl.BlockSpec((1, row_block, KNN), lambda b, i: (b, i, 0)),
        out_shape=jax.ShapeDtypeStruct((B, N, KNN), jnp.int32),
        interpret=interpret,
    )(xt, xt)


def _sc_gather(x, idx_flat):
    B, C, N = x.shape
    NK = N * KNN
    info = plsc.get_sparse_core_info()
    NC, NS = info.num_cores, info.num_subcores
    NW = NC * NS
    wpb = NW // B
    cpw = C // wpb
    mesh = plsc.VectorSubcoreMesh(core_axis_name="c", subcore_axis_name="s")

    @functools.partial(
        pl.kernel,
        out_type=jax.ShapeDtypeStruct((B, 2 * C, NK), jnp.float32),
        mesh=mesh,
        compiler_params=pltpu.CompilerParams(needs_layout_passes=False),
        scratch_types=[
            pltpu.VMEM((NK,), jnp.int32),
            pltpu.VMEM((N,), jnp.float32),
            pltpu.VMEM((NK,), jnp.float32),
            pltpu.VMEM((NK,), jnp.float32),
        ],
    )
    def sc_kernel(x_hbm, idx_hbm, out_hbm, idx_v, row_v, cbuf, nbuf):
        wid = lax.axis_index("s") * NC + lax.axis_index("c")
        b = wid // wpb
        c0 = (wid % wpb) * cpw
        pltpu.sync_copy(idx_hbm.at[b], idx_v)

        def chan(ch, _):
            c = c0 + ch
            pltpu.sync_copy(x_hbm.at[b, c], row_v)

            def point(i, _):
                iv = idx_v[pl.ds(i * KNN, KNN)]
                nb = plsc.load_gather(row_v, [iv])
                ci = lax.broadcast_in_dim(i, (KNN,), ())
                center = plsc.load_gather(row_v, [ci])
                cbuf[pl.ds(i * KNN, KNN)] = center
                nbuf[pl.ds(i * KNN, KNN)] = nb - center
                return 0

            lax.fori_loop(0, N, point, 0)
            pltpu.sync_copy(cbuf, out_hbm.at[b, c])
            pltpu.sync_copy(nbuf, out_hbm.at[b, C + c])
            return 0

        lax.fori_loop(0, cpw, chan, 0)

    return sc_kernel(x, idx_flat)


def kernel(cloud):
    B, C, N = cloud.shape
    xt = jnp.transpose(cloud, (0, 2, 1))
    idx = _topk_indices(xt, row_block=256)
    idx_flat = idx.reshape(B, N * KNN)
    out = _sc_gather(cloud, idx_flat)
    return out.reshape(B, 2 * C, N, KNN)
